# baseline (speedup 1.0000x reference)
import functools
import jax
import jax.numpy as jnp
from jax import lax
from jax.experimental import pallas as pl

B, N, D, TD, K = 4, 1024, 256, 256, 16
MB = 64
MA = 512
TW = 384

_f32 = jnp.float32


def _topk_body(pos_ref, idx_ref):
    p = pos_ref[0]
    pt = p.T
    s_col = jnp.sum(p * p, axis=-1, keepdims=True)
    s_row = jnp.sum(pt * pt, axis=0, keepdims=True)
    g = jnp.dot(p, pt, preferred_element_type=_f32)
    d = s_col - 2.0 * g + s_row
    iota = lax.broadcasted_iota(jnp.int32, (N, N), 1)
    cols = []
    for _ in range(K):
        m = jnp.min(d, axis=1, keepdims=True)
        cand = jnp.where(d <= m, iota, jnp.int32(2**30))
        idx = jnp.min(cand, axis=1, keepdims=True)
        cols.append(idx)
        d = jnp.where(iota == idx, jnp.float32(jnp.inf), d)
    idx_ref[0] = jnp.concatenate(cols, axis=1)


def _prep_body(x_ref, pos_ref, wfc1_ref, bfc1_ref, wq_ref, wg1_ref,
               wd1_ref, qg1_ref, pw_ref, tab_ref):
    x = x_ref[0]
    h = jnp.dot(x, wfc1_ref[...], preferred_element_type=_f32) + bfc1_ref[...]
    wqg1 = jnp.dot(wq_ref[...], wg1_ref[...], preferred_element_type=_f32)
    qg1_ref[0] = jnp.dot(h, wqg1, preferred_element_type=_f32)
    p = pos_ref[0]
    pw_ref[0] = jnp.dot(p, wd1_ref[...], preferred_element_type=_f32)
    pad = jnp.zeros((MA, TW - D - 3), _f32)
    tab_ref[0] = jnp.concatenate([h, p, pad], axis=-1)


def _fused_body(idx_ref, qg1_ref, pw_ref, x_ref, tab_ref,
                wk_ref, wg1_ref, wd1_ref,
                bd1_ref, wd2_ref, bd2_ref, bg1_ref,
                wg2_ref, bg2_ref, wv_ref, wfc2_ref, bfc2_ref,
                res_ref, attn_ref):
    idx = idx_ref[0]
    iota3 = lax.broadcasted_iota(jnp.int32, (MB, K, N), 2)
    oh = (idx[:, :, None] == iota3).astype(_f32)
    ohf = oh.reshape(MB * K, N)
    g = jnp.dot(ohf, tab_ref[0], preferred_element_type=_f32)
    hg = g[:, :D]
    posg = g[:, D:D + 128][:, :3]

    pwg = jnp.dot(posg, wd1_ref[...], preferred_element_type=_f32)
    pwb = pw_ref[0]
    r_in = pwb[:, None, :] - pwg.reshape(MB, K, TD) + bd1_ref[...]
    r = jnp.maximum(r_in, 0.0).reshape(MB * K, TD)
    pe = jnp.dot(r, wd2_ref[...], preferred_element_type=_f32) + bd2_ref[...]

    wkg1 = jnp.dot(wk_ref[...], wg1_ref[...], preferred_element_type=_f32)
    kg1 = jnp.dot(hg, wkg1, preferred_element_type=_f32)
    peg1 = jnp.dot(pe, wg1_ref[...], preferred_element_type=_f32)
    pre1 = qg1_ref[0][:, None, :] - kg1.reshape(MB, K, TD) \
        + peg1.reshape(MB, K, TD) + bg1_ref[...]
    a1 = jnp.maximum(pre1, 0.0).reshape(MB * K, TD)
    logits = jnp.dot(a1, wg2_ref[...], preferred_element_type=_f32) + bg2_ref[...]
    l3 = logits.reshape(MB, K, TD) * jnp.float32(1.0 / 16.0)

    mx = jnp.max(l3, axis=1, keepdims=True)
    e = jnp.exp(l3 - mx)
    s = jnp.sum(e, axis=1, keepdims=True)
    attn = e / s
    attn_ref[0] = attn

    vg = jnp.dot(hg, wv_ref[...], preferred_element_type=_f32)
    wsum = attn * (vg.reshape(MB, K, TD) + pe.reshape(MB, K, TD))
    rsum = jnp.sum(wsum, axis=1)
    out = jnp.dot(rsum, wfc2_ref[...], preferred_element_type=_f32) \
        + bfc2_ref[...] + x_ref[0]
    res_ref[0] = out


def kernel(x, pos, W_fc1, b_fc1, W_fc2, b_fc2, W_d1, b_d1, W_d2, b_d2,
           W_g1, b_g1, W_g2, b_g2, W_q, W_k, W_v):
    b_fc1r = b_fc1.reshape(1, TD)
    b_fc2r = b_fc2.reshape(1, D)
    b_d1r = b_d1.reshape(1, TD)
    b_d2r = b_d2.reshape(1, TD)
    b_g1r = b_g1.reshape(1, TD)
    b_g2r = b_g2.reshape(1, TD)

    knn_idx = pl.pallas_call(
        _topk_body,
        grid=(B,),
        in_specs=[pl.BlockSpec((1, N, 3), lambda b: (b, 0, 0))],
        out_specs=pl.BlockSpec((1, N, K), lambda b: (b, 0, 0)),
        out_shape=jax.ShapeDtypeStruct((B, N, K), jnp.int32),
    )(pos)

    full = lambda shp: pl.BlockSpec(shp, lambda b, m: tuple(0 for _ in shp))
    qg1, pw, tab = pl.pallas_call(
        _prep_body,
        grid=(B, N // MA),
        in_specs=[
            pl.BlockSpec((1, MA, D), lambda b, m: (b, m, 0)),
            pl.BlockSpec((1, MA, 3), lambda b, m: (b, m, 0)),
            full((D, TD)), full((1, TD)),
            full((TD, TD)), full((TD, TD)),
            full((3, TD)),
        ],
        out_specs=[
            pl.BlockSpec((1, MA, TD), lambda b, m: (b, m, 0)),
            pl.BlockSpec((1, MA, TD), lambda b, m: (b, m, 0)),
            pl.BlockSpec((1, MA, TW), lambda b, m: (b, m, 0)),
        ],
        out_shape=[
            jax.ShapeDtypeStruct((B, N, TD), _f32),
            jax.ShapeDtypeStruct((B, N, TD), _f32),
            jax.ShapeDtypeStruct((B, N, TW), _f32),
        ],
    )(x, pos, W_fc1, b_fc1r, W_q, W_g1, W_d1)

    res, attn = pl.pallas_call(
        _fused_body,
        grid=(B, N // MB),
        in_specs=[
            pl.BlockSpec((1, MB, K), lambda b, m: (b, m, 0)),
            pl.BlockSpec((1, MB, TD), lambda b, m: (b, m, 0)),
            pl.BlockSpec((1, MB, TD), lambda b, m: (b, m, 0)),
            pl.BlockSpec((1, MB, D), lambda b, m: (b, m, 0)),
            pl.BlockSpec((1, N, TW), lambda b, m: (b, 0, 0)),
            full((TD, TD)), full((TD, TD)), full((3, TD)),
            full((1, TD)), full((TD, TD)), full((1, TD)), full((1, TD)),
            full((TD, TD)), full((1, TD)), full((TD, TD)),
            full((TD, D)), full((1, D)),
        ],
        out_specs=[
            pl.BlockSpec((1, MB, D), lambda b, m: (b, m, 0)),
            pl.BlockSpec((1, MB, K, TD), lambda b, m: (b, m, 0, 0)),
        ],
        out_shape=[
            jax.ShapeDtypeStruct((B, N, D), _f32),
            jax.ShapeDtypeStruct((B, N, K, TD), _f32),
        ],
    )(knn_idx, qg1, pw, x, tab,
      W_k, W_g1, W_d1, b_d1r, W_d2, b_d2r, b_g1r, W_g2, b_g2r, W_v,
      W_fc2, b_fc2r)

    return (res, attn)
